# baseline (speedup 1.0000x reference)
import dataclasses
import functools

import jax
import jax.numpy as jnp
from jax import lax
from jax.experimental import pallas as pl
from jax.experimental.pallas import tpu as pltpu
from jax.experimental.pallas import tpu_sc as plsc

N = 10000
C = 128
E = 320000
EPS = 0.1

NC = 2
NS = 16
NW = NC * NS
BLK = 64
NBLK = E // BLK
NB_LO = NBLK // NW
NB_EXTRA = NBLK - NB_LO * NW
EPW_HI = (NB_LO + 1) * BLK
EPW_LO = NB_LO * BLK
RPW = 624
LANES = 16

_mesh = plsc.VectorSubcoreMesh(core_axis_name="c", subcore_axis_name="s")


def _sc_compiler_params():
    cp = pltpu.CompilerParams()
    if "needs_layout_passes" in pltpu.CompilerParams.__dataclass_fields__:
        cp = dataclasses.replace(cp, needs_layout_passes=False)
    return cp


def _sc_body(x_hbm, row_hbm, col_hbm, pq_hbm, a_out, s_out,
             rows_all, cols_all, zrow,
             xbuf0, xbuf1, xbuf2, pqb0, pqb1, pqb2,
             idx20, idx21, idx22, cols20, cols21, cols22,
             wbuf0, wbuf1, wbuf2,
             a_sh, s_sh, gsem0, gsem1, gsem2, ssem0, ssem1, ssem2):
    c = lax.axis_index("c")
    s_ = lax.axis_index("s")
    wid = s_ * NC + c
    ext = wid < NB_EXTRA
    nb_eff = jnp.where(ext, NB_LO + 1, NB_LO)
    ebase0 = jnp.where(ext, wid * EPW_HI, EPW_LO * wid + NB_EXTRA * BLK)

    @pl.when(ext)
    def _():
        pltpu.sync_copy(row_hbm.at[pl.ds(ebase0, EPW_HI)], rows_all)
        pltpu.sync_copy(col_hbm.at[pl.ds(ebase0, EPW_HI)], cols_all)

    @pl.when(jnp.logical_not(ext))
    def _():
        pltpu.sync_copy(row_hbm.at[pl.ds(ebase0, EPW_LO)],
                        rows_all.at[pl.ds(0, EPW_LO)])
        pltpu.sync_copy(col_hbm.at[pl.ds(ebase0, EPW_LO)],
                        cols_all.at[pl.ds(0, EPW_LO)])

    bufs = ((xbuf0, pqb0, idx20, cols20, wbuf0, gsem0, ssem0),
            (xbuf1, pqb1, idx21, cols21, wbuf1, gsem1, ssem1),
            (xbuf2, pqb2, idx22, cols22, wbuf2, gsem2, ssem2))

    HB = BLK // 2

    def prep_idx(j, b):
        _, _, idx2, cols2, _, _, _ = bufs[b]
        for jj in range(BLK // LANES):
            sl = pl.ds(jj * LANES, LANES)
            rv = rows_all[pl.ds(j * BLK + jj * LANES, LANES)]
            cv = cols_all[pl.ds(j * BLK + jj * LANES, LANES)]
            idx2[0, sl] = rv
            idx2[0, pl.ds(BLK + jj * LANES, LANES)] = cv + N
            cols2[0, sl] = cv

    def launch(j, b):
        xbuf, pqb, idx2, _, _, gsem, _ = bufs[b]
        prep_idx(j, b)
        pltpu.async_copy(x_hbm.at[rows_all.at[pl.ds(j * BLK, HB)]],
                         xbuf.at[pl.ds(0, HB)], gsem)
        pltpu.async_copy(x_hbm.at[rows_all.at[pl.ds(j * BLK + HB, HB)]],
                         xbuf.at[pl.ds(HB, HB)], gsem)
        pltpu.async_copy(pq_hbm.at[idx2.at[0]], pqb, gsem)

    launch(0, 0)

    zz = jnp.zeros((LANES,), jnp.float32)

    @pl.loop(0, BLK)
    def _(i):
        for j in range(C // LANES):
            xbuf1[i, pl.ds(j * LANES, LANES)] = zz

    @pl.loop(0, RPW // LANES)
    def _(i):
        zrow[pl.ds(i * LANES, LANES)] = zz

    arow0 = s_ * RPW

    def for_owned_chunks(fn):
        for k in range(RPW // BLK):
            fn(arow0 + k * BLK, BLK)
        fn(arow0 + (RPW // BLK) * BLK, RPW - (RPW // BLK) * BLK)

        @pl.when(s_ == NS - 1)
        def _():
            fn(NS * RPW, N - NS * RPW)

    for_owned_chunks(
        lambda start, size: pltpu.sync_copy(
            xbuf1.at[pl.ds(0, size)], a_sh.at[pl.ds(start, size)]))
    pltpu.sync_copy(zrow, s_sh.at[pl.ds(arow0, RPW)])

    @pl.when(s_ == NS - 1)
    def _():
        pltpu.sync_copy(zrow.at[pl.ds(0, N - NS * RPW)],
                        s_sh.at[pl.ds(NS * RPW, N - NS * RPW)])

    plsc.subcore_barrier()

    def drain_gathers(b):
        xbuf, pqb, _, _, _, gsem, _ = bufs[b]
        pltpu.make_async_copy(x_hbm.at[pl.ds(0, BLK)], xbuf, gsem).wait()
        pltpu.make_async_copy(pq_hbm.at[pl.ds(0, 2 * BLK)], pqb, gsem).wait()

    def drain_scatters(b):
        xbuf, _, _, _, wbuf, _, ssem = bufs[b]
        pltpu.make_async_copy(x_hbm.at[pl.ds(0, BLK)], xbuf, ssem).wait()
        pltpu.make_async_copy(pq_hbm.at[pl.ds(0, BLK)], wbuf, ssem).wait()

    def process(j, b, prefetch=True, drain_prev=True):
        xbuf, pqb, idx2, cols2, wbuf, gsem, ssem = bufs[b]
        bn = (b + 1) % 3

        if drain_prev:
            drain_scatters(bn)
        if prefetch:
            @pl.when(j + 1 < nb_eff)
            def _():
                launch(j + 1, bn)

        drain_gathers(b)
        for jj in range(BLK // LANES):
            sl = pl.ds(jj * LANES, LANES)
            z = pqb[sl] + pqb[pl.ds(BLK + jj * LANES, LANES)]
            t = jnp.exp(jnp.abs(z) * 2.0)
            th = jnp.sign(z) * (1.0 - 2.0 / (t + 1.0))
            wbuf[sl] = jnp.exp(th)
        pltpu.async_copy(wbuf, s_sh.at[cols2.at[0]], ssem, add=True)

        @pl.loop(0, BLK)
        def _(i):
            wv = plsc.load_gather(wbuf, [jnp.full((LANES,), i, jnp.int32)])
            for jj in range(C // LANES):
                xbuf[i, pl.ds(jj * LANES, LANES)] = (
                    xbuf[i, pl.ds(jj * LANES, LANES)] * wv)

        pltpu.async_copy(xbuf, a_sh.at[cols2.at[0]], ssem, add=True)

    process(0, 0, drain_prev=False)
    process(1, 1, drain_prev=False)

    @pl.loop(2, NB_LO - 4, step=3)
    def _(g):
        for db in range(3):
            j = g + db
            process(j, (2 + db) % 3)

    for j in range(NB_LO - 4, NB_LO):
        process(j, j % 3)

    @pl.when(ext)
    def _():
        process(NB_LO, NB_LO % 3, prefetch=False)

    @pl.when(ext)
    def _():
        drain_scatters((NB_LO - 1) % 3)
        drain_scatters(NB_LO % 3)

    @pl.when(jnp.logical_not(ext))
    def _():
        drain_scatters((NB_LO - 2) % 3)
        drain_scatters((NB_LO - 1) % 3)

    plsc.subcore_barrier()

    for_owned_chunks(
        lambda start, size: pltpu.sync_copy(
            a_sh.at[pl.ds(start, size)], a_out.at[c, pl.ds(start, size)]))

    pltpu.sync_copy(s_sh.at[pl.ds(arow0, RPW)], zrow)
    pltpu.sync_copy(zrow, s_out.at[pl.ds(c * N + arow0, RPW)])

    @pl.when(s_ == NS - 1)
    def _():
        pltpu.sync_copy(s_sh.at[pl.ds(NS * RPW, N - NS * RPW)],
                        zrow.at[pl.ds(0, N - NS * RPW)])
        pltpu.sync_copy(zrow.at[pl.ds(0, N - NS * RPW)],
                        s_out.at[pl.ds(c * N + NS * RPW, N - NS * RPW)])


def _sc_edge_pass(x, row, col, pq):
    f = pl.kernel(
        _sc_body,
        out_type=[
            jax.ShapeDtypeStruct((NC, N, C), jnp.float32),
            jax.ShapeDtypeStruct((NC * N,), jnp.float32),
        ],
        mesh=_mesh,
        scratch_types=(
            [
                pltpu.VMEM((EPW_HI,), jnp.int32),
                pltpu.VMEM((EPW_HI,), jnp.int32),
                pltpu.VMEM((RPW,), jnp.float32),
            ]
            + [pltpu.VMEM((BLK, C), jnp.float32)] * 3
            + [pltpu.VMEM((2 * BLK,), jnp.float32)] * 3
            + [pltpu.VMEM((1, 2 * BLK), jnp.int32)] * 3
            + [pltpu.VMEM((1, BLK), jnp.int32)] * 3
            + [pltpu.VMEM((BLK,), jnp.float32)] * 3
            + [
                pltpu.VMEM_SHARED((N, C), jnp.float32),
                pltpu.VMEM_SHARED((N,), jnp.float32),
            ]
            + [pltpu.SemaphoreType.DMA] * 6
        ),
        compiler_params=_sc_compiler_params(),
    )
    return f(x, row, col, pq)


def _pq_body(w2_ref, x_ref, b_ref, o_ref):
    o_ref[...] = lax.dot_general(
        w2_ref[...], x_ref[...], (((1,), (1,)), ((), ())),
        preferred_element_type=jnp.float32) + b_ref[...]


def _pq_pass(x, w2t, bvec):
    return pl.pallas_call(
        _pq_body,
        out_shape=jax.ShapeDtypeStruct((2, N), jnp.float32),
    )(w2t, x, bvec)


def _fin_body(a_ref, s_ref, x_ref, w_ref, o_ref):
    A = a_ref[0] + a_ref[1]
    sv = jnp.sum(s_ref[...], axis=0) + 1e-16
    Y = lax.dot_general(A, w_ref[...], (((1,), (1,)), ((), ())),
                        preferred_element_type=jnp.float32)
    o_ref[...] = (1.0 - EPS) * (Y / sv[:, None]) + EPS * x_ref[...]


def _fin_pass(a_parts, s_parts, x, W_msg):
    return pl.pallas_call(
        _fin_body,
        out_shape=jax.ShapeDtypeStruct((N, C), jnp.float32),
    )(a_parts, s_parts, x, W_msg)


@jax.jit
def kernel(x, edge_index, W_att, b_att, W_msg):
    row = edge_index[0]
    col = edge_index[1]
    w2t = W_att.reshape(2, C)
    bvec = jnp.concatenate([jnp.zeros((1,), jnp.float32), b_att]).reshape(2, 1)
    pq = _pq_pass(x, w2t, bvec)
    a_parts, s_parts = _sc_edge_pass(x, row, col, pq.reshape(2 * N))
    return _fin_pass(a_parts, s_parts.reshape(NC, N), x, W_msg)


# speedup vs baseline: 1.0636x; 1.0636x over previous
import dataclasses
import functools

import jax
import jax.numpy as jnp
from jax import lax
from jax.experimental import pallas as pl
from jax.experimental.pallas import tpu as pltpu
from jax.experimental.pallas import tpu_sc as plsc

N = 10000
C = 128
E = 320000
EPS = 0.1

NC = 2
NS = 16
NW = NC * NS
BLK = 64
NBLK = E // BLK
NB_LO = NBLK // NW
NB_EXTRA = NBLK - NB_LO * NW
EPW_HI = (NB_LO + 1) * BLK
EPW_LO = NB_LO * BLK
RPW = 624
LANES = 16

_mesh = plsc.VectorSubcoreMesh(core_axis_name="c", subcore_axis_name="s")


def _sc_compiler_params():
    cp = pltpu.CompilerParams()
    if "needs_layout_passes" in pltpu.CompilerParams.__dataclass_fields__:
        cp = dataclasses.replace(cp, needs_layout_passes=False)
    return cp


def _sc_body(x_hbm, row_hbm, col_hbm, pq_hbm, a_out, s_out,
             rows_all, cols_all, zrow,
             xbuf0, xbuf1, xbuf2, pqb0, pqb1, pqb2,
             idx20, idx21, idx22, cols20, cols21, cols22,
             wbuf0, wbuf1, wbuf2,
             a_sh, s_sh, gsem0, gsem1, gsem2, ssem0, ssem1, ssem2):
    c = lax.axis_index("c")
    s_ = lax.axis_index("s")
    wid = s_ * NC + c
    ext = wid < NB_EXTRA
    nb_eff = jnp.where(ext, NB_LO + 1, NB_LO)
    ebase0 = jnp.where(ext, wid * EPW_HI, EPW_LO * wid + NB_EXTRA * BLK)

    @pl.when(ext)
    def _():
        pltpu.sync_copy(row_hbm.at[pl.ds(ebase0, EPW_HI)], rows_all)
        pltpu.sync_copy(col_hbm.at[pl.ds(ebase0, EPW_HI)], cols_all)

    @pl.when(jnp.logical_not(ext))
    def _():
        pltpu.sync_copy(row_hbm.at[pl.ds(ebase0, EPW_LO)],
                        rows_all.at[pl.ds(0, EPW_LO)])
        pltpu.sync_copy(col_hbm.at[pl.ds(ebase0, EPW_LO)],
                        cols_all.at[pl.ds(0, EPW_LO)])

    bufs = ((xbuf0, pqb0, idx20, cols20, wbuf0, gsem0, ssem0),
            (xbuf1, pqb1, idx21, cols21, wbuf1, gsem1, ssem1),
            (xbuf2, pqb2, idx22, cols22, wbuf2, gsem2, ssem2))

    HB = BLK // 2

    def prep_idx(j, b):
        _, _, idx2, cols2, _, _, _ = bufs[b]
        for jj in range(BLK // LANES):
            sl = pl.ds(jj * LANES, LANES)
            rv = rows_all[pl.ds(j * BLK + jj * LANES, LANES)]
            cv = cols_all[pl.ds(j * BLK + jj * LANES, LANES)]
            idx2[0, sl] = rv
            idx2[0, pl.ds(BLK + jj * LANES, LANES)] = cv + N
            cols2[0, sl] = cv

    def launch(j, b):
        xbuf, pqb, idx2, _, _, gsem, _ = bufs[b]
        prep_idx(j, b)
        pltpu.async_copy(x_hbm.at[rows_all.at[pl.ds(j * BLK, BLK)]], xbuf, gsem)
        pltpu.async_copy(pq_hbm.at[idx2.at[0]], pqb, gsem)

    launch(0, 0)

    zz = jnp.zeros((LANES,), jnp.float32)

    @pl.loop(0, BLK)
    def _(i):
        for j in range(C // LANES):
            xbuf1[i, pl.ds(j * LANES, LANES)] = zz

    @pl.loop(0, RPW // LANES)
    def _(i):
        zrow[pl.ds(i * LANES, LANES)] = zz

    arow0 = s_ * RPW

    def for_owned_chunks(fn):
        for k in range(RPW // BLK):
            fn(arow0 + k * BLK, BLK)
        fn(arow0 + (RPW // BLK) * BLK, RPW - (RPW // BLK) * BLK)

        @pl.when(s_ == NS - 1)
        def _():
            fn(NS * RPW, N - NS * RPW)

    for_owned_chunks(
        lambda start, size: pltpu.sync_copy(
            xbuf1.at[pl.ds(0, size)], a_sh.at[pl.ds(start, size)]))
    pltpu.sync_copy(zrow, s_sh.at[pl.ds(arow0, RPW)])

    @pl.when(s_ == NS - 1)
    def _():
        pltpu.sync_copy(zrow.at[pl.ds(0, N - NS * RPW)],
                        s_sh.at[pl.ds(NS * RPW, N - NS * RPW)])

    plsc.subcore_barrier()

    def drain_gathers(b):
        xbuf, pqb, _, _, _, gsem, _ = bufs[b]
        pltpu.make_async_copy(x_hbm.at[pl.ds(0, BLK)], xbuf, gsem).wait()
        pltpu.make_async_copy(pq_hbm.at[pl.ds(0, 2 * BLK)], pqb, gsem).wait()

    def drain_scatters(b):
        xbuf, _, _, _, wbuf, _, ssem = bufs[b]
        pltpu.make_async_copy(x_hbm.at[pl.ds(0, BLK)], xbuf, ssem).wait()
        pltpu.make_async_copy(pq_hbm.at[pl.ds(0, BLK)], wbuf, ssem).wait()

    def process(j, b, prefetch=True, drain_prev=True):
        xbuf, pqb, idx2, cols2, wbuf, gsem, ssem = bufs[b]
        bn = (b + 1) % 3

        if drain_prev:
            drain_scatters(bn)
        if prefetch:
            @pl.when(j + 1 < nb_eff)
            def _():
                launch(j + 1, bn)

        drain_gathers(b)
        for jj in range(BLK // LANES):
            sl = pl.ds(jj * LANES, LANES)
            z = pqb[sl] + pqb[pl.ds(BLK + jj * LANES, LANES)]
            t = jnp.exp(jnp.abs(z) * 2.0)
            th = jnp.sign(z) * (1.0 - 2.0 / (t + 1.0))
            wbuf[sl] = jnp.exp(th)
        pltpu.async_copy(wbuf, s_sh.at[cols2.at[0]], ssem, add=True)

        @pl.loop(0, BLK, step=2)
        def _(i):
            wv0 = plsc.load_gather(wbuf, [jnp.full((LANES,), i, jnp.int32)])
            wv1 = plsc.load_gather(wbuf, [jnp.full((LANES,), i + 1, jnp.int32)])
            for jj in range(C // LANES):
                xbuf[i, pl.ds(jj * LANES, LANES)] = (
                    xbuf[i, pl.ds(jj * LANES, LANES)] * wv0)
            for jj in range(C // LANES):
                xbuf[i + 1, pl.ds(jj * LANES, LANES)] = (
                    xbuf[i + 1, pl.ds(jj * LANES, LANES)] * wv1)

        pltpu.async_copy(xbuf, a_sh.at[cols2.at[0]], ssem, add=True)

    process(0, 0, drain_prev=False)
    process(1, 1, drain_prev=False)

    @pl.loop(2, NB_LO - 4, step=3)
    def _(g):
        for db in range(3):
            j = g + db
            process(j, (2 + db) % 3)

    for j in range(NB_LO - 4, NB_LO):
        process(j, j % 3)

    @pl.when(ext)
    def _():
        process(NB_LO, NB_LO % 3, prefetch=False)

    @pl.when(ext)
    def _():
        drain_scatters((NB_LO - 1) % 3)
        drain_scatters(NB_LO % 3)

    @pl.when(jnp.logical_not(ext))
    def _():
        drain_scatters((NB_LO - 2) % 3)
        drain_scatters((NB_LO - 1) % 3)

    plsc.subcore_barrier()

    for_owned_chunks(
        lambda start, size: pltpu.sync_copy(
            a_sh.at[pl.ds(start, size)], a_out.at[c, pl.ds(start, size)]))

    pltpu.sync_copy(s_sh.at[pl.ds(arow0, RPW)], zrow)
    pltpu.sync_copy(zrow, s_out.at[pl.ds(c * N + arow0, RPW)])

    @pl.when(s_ == NS - 1)
    def _():
        pltpu.sync_copy(s_sh.at[pl.ds(NS * RPW, N - NS * RPW)],
                        zrow.at[pl.ds(0, N - NS * RPW)])
        pltpu.sync_copy(zrow.at[pl.ds(0, N - NS * RPW)],
                        s_out.at[pl.ds(c * N + NS * RPW, N - NS * RPW)])


def _sc_edge_pass(x, row, col, pq):
    f = pl.kernel(
        _sc_body,
        out_type=[
            jax.ShapeDtypeStruct((NC, N, C), jnp.float32),
            jax.ShapeDtypeStruct((NC * N,), jnp.float32),
        ],
        mesh=_mesh,
        scratch_types=(
            [
                pltpu.VMEM((EPW_HI,), jnp.int32),
                pltpu.VMEM((EPW_HI,), jnp.int32),
                pltpu.VMEM((RPW,), jnp.float32),
            ]
            + [pltpu.VMEM((BLK, C), jnp.float32)] * 3
            + [pltpu.VMEM((2 * BLK,), jnp.float32)] * 3
            + [pltpu.VMEM((1, 2 * BLK), jnp.int32)] * 3
            + [pltpu.VMEM((1, BLK), jnp.int32)] * 3
            + [pltpu.VMEM((BLK,), jnp.float32)] * 3
            + [
                pltpu.VMEM_SHARED((N, C), jnp.float32),
                pltpu.VMEM_SHARED((N,), jnp.float32),
            ]
            + [pltpu.SemaphoreType.DMA] * 6
        ),
        compiler_params=_sc_compiler_params(),
    )
    return f(x, row, col, pq)


def _pq_body(w2_ref, x_ref, b_ref, o_ref):
    o_ref[...] = lax.dot_general(
        w2_ref[...], x_ref[...], (((1,), (1,)), ((), ())),
        preferred_element_type=jnp.float32) + b_ref[...]


def _pq_pass(x, w2t, bvec):
    return pl.pallas_call(
        _pq_body,
        out_shape=jax.ShapeDtypeStruct((2, N), jnp.float32),
    )(w2t, x, bvec)


def _fin_body(a_ref, s_ref, x_ref, w_ref, o_ref):
    A = a_ref[0] + a_ref[1]
    sv = jnp.sum(s_ref[...], axis=0) + 1e-16
    Y = lax.dot_general(A, w_ref[...], (((1,), (1,)), ((), ())),
                        preferred_element_type=jnp.float32)
    o_ref[...] = (1.0 - EPS) * (Y / sv[:, None]) + EPS * x_ref[...]


def _fin_pass(a_parts, s_parts, x, W_msg):
    return pl.pallas_call(
        _fin_body,
        out_shape=jax.ShapeDtypeStruct((N, C), jnp.float32),
    )(a_parts, s_parts, x, W_msg)


@jax.jit
def kernel(x, edge_index, W_att, b_att, W_msg):
    row = edge_index[0]
    col = edge_index[1]
    w2t = W_att.reshape(2, C)
    bvec = jnp.concatenate([jnp.zeros((1,), jnp.float32), b_att]).reshape(2, 1)
    pq = _pq_pass(x, w2t, bvec)
    a_parts, s_parts = _sc_edge_pass(x, row, col, pq.reshape(2 * N))
    return _fin_pass(a_parts, s_parts.reshape(NC, N), x, W_msg)


# speedup vs baseline: 1.0864x; 1.0214x over previous
import dataclasses
import functools

import jax
import jax.numpy as jnp
from jax import lax
from jax.experimental import pallas as pl
from jax.experimental.pallas import tpu as pltpu
from jax.experimental.pallas import tpu_sc as plsc

N = 10000
C = 128
E = 320000
EPS = 0.1

NC = 2
NS = 16
NW = NC * NS
BLK = 64
NBLK = E // BLK
NB_LO = NBLK // NW
NB_EXTRA = NBLK - NB_LO * NW
EPW_HI = (NB_LO + 1) * BLK
EPW_LO = NB_LO * BLK
RPW = 624
LANES = 16

_mesh = plsc.VectorSubcoreMesh(core_axis_name="c", subcore_axis_name="s")


def _sc_compiler_params():
    cp = pltpu.CompilerParams()
    if "needs_layout_passes" in pltpu.CompilerParams.__dataclass_fields__:
        cp = dataclasses.replace(cp, needs_layout_passes=False)
    return cp


def _sc_body(x_hbm, row_hbm, col_hbm, pq_hbm, a_out, s_out,
             rows_all, cols_all, zrow,
             xbuf0, xbuf1, xbuf2, pqb0, pqb1, pqb2,
             idx20, idx21, idx22, cols20, cols21, cols22,
             wbuf0, wbuf1, wbuf2,
             a_sh, s_sh, gsem0, gsem1, gsem2, ssem0, ssem1, ssem2):
    c = lax.axis_index("c")
    s_ = lax.axis_index("s")
    wid = s_ * NC + c
    ext = wid < NB_EXTRA
    nb_eff = jnp.where(ext, NB_LO + 1, NB_LO)
    ebase0 = jnp.where(ext, wid * EPW_HI, EPW_LO * wid + NB_EXTRA * BLK)

    @pl.when(ext)
    def _():
        pltpu.sync_copy(row_hbm.at[pl.ds(ebase0, EPW_HI)], rows_all)
        pltpu.sync_copy(col_hbm.at[pl.ds(ebase0, EPW_HI)], cols_all)

    @pl.when(jnp.logical_not(ext))
    def _():
        pltpu.sync_copy(row_hbm.at[pl.ds(ebase0, EPW_LO)],
                        rows_all.at[pl.ds(0, EPW_LO)])
        pltpu.sync_copy(col_hbm.at[pl.ds(ebase0, EPW_LO)],
                        cols_all.at[pl.ds(0, EPW_LO)])

    bufs = ((xbuf0, pqb0, idx20, cols20, wbuf0, gsem0, ssem0),
            (xbuf1, pqb1, idx21, cols21, wbuf1, gsem1, ssem1),
            (xbuf2, pqb2, idx22, cols22, wbuf2, gsem2, ssem2))

    HB = BLK // 2

    def prep_idx(j, b):
        _, _, idx2, cols2, _, _, _ = bufs[b]
        for jj in range(BLK // LANES):
            sl = pl.ds(jj * LANES, LANES)
            rv = rows_all[pl.ds(j * BLK + jj * LANES, LANES)]
            cv = cols_all[pl.ds(j * BLK + jj * LANES, LANES)]
            idx2[0, sl] = rv
            idx2[0, pl.ds(BLK + jj * LANES, LANES)] = cv + N
            cols2[0, sl] = cv

    def launch(j, b):
        xbuf, pqb, idx2, _, _, gsem, _ = bufs[b]
        prep_idx(j, b)
        pltpu.async_copy(x_hbm.at[rows_all.at[pl.ds(j * BLK, BLK)]], xbuf, gsem)
        pltpu.async_copy(pq_hbm.at[idx2.at[0]], pqb, gsem)

    launch(0, 0)

    zz = jnp.zeros((LANES,), jnp.float32)

    @pl.loop(0, BLK)
    def _(i):
        for j in range(C // LANES):
            xbuf1[i, pl.ds(j * LANES, LANES)] = zz

    @pl.loop(0, RPW // LANES)
    def _(i):
        zrow[pl.ds(i * LANES, LANES)] = zz

    arow0 = s_ * RPW

    def for_owned_chunks(fn):
        for k in range(RPW // BLK):
            fn(arow0 + k * BLK, BLK)
        fn(arow0 + (RPW // BLK) * BLK, RPW - (RPW // BLK) * BLK)

        @pl.when(s_ == NS - 1)
        def _():
            fn(NS * RPW, N - NS * RPW)

    for_owned_chunks(
        lambda start, size: pltpu.sync_copy(
            xbuf1.at[pl.ds(0, size)], a_sh.at[pl.ds(start, size)]))
    pltpu.sync_copy(zrow, s_sh.at[pl.ds(arow0, RPW)])

    @pl.when(s_ == NS - 1)
    def _():
        pltpu.sync_copy(zrow.at[pl.ds(0, N - NS * RPW)],
                        s_sh.at[pl.ds(NS * RPW, N - NS * RPW)])

    plsc.subcore_barrier()

    def drain_gathers(b):
        xbuf, pqb, _, _, _, gsem, _ = bufs[b]
        pltpu.make_async_copy(x_hbm.at[pl.ds(0, BLK)], xbuf, gsem).wait()
        pltpu.make_async_copy(pq_hbm.at[pl.ds(0, 2 * BLK)], pqb, gsem).wait()

    def drain_scatters(b):
        xbuf, _, _, _, wbuf, _, ssem = bufs[b]
        pltpu.make_async_copy(x_hbm.at[pl.ds(0, BLK)], xbuf, ssem).wait()
        pltpu.make_async_copy(pq_hbm.at[pl.ds(0, BLK)], wbuf, ssem).wait()

    def process(j, b, prefetch=True, drain_prev=True):
        xbuf, pqb, idx2, cols2, wbuf, gsem, ssem = bufs[b]
        bn = (b + 1) % 3

        if drain_prev:
            drain_scatters(bn)
        if prefetch:
            @pl.when(j + 1 < nb_eff)
            def _():
                launch(j + 1, bn)

        drain_gathers(b)
        for jj in range(BLK // LANES):
            sl = pl.ds(jj * LANES, LANES)
            z = pqb[sl] + pqb[pl.ds(BLK + jj * LANES, LANES)]
            t = jnp.exp(jnp.abs(z) * 2.0)
            th = jnp.sign(z) * (1.0 - 2.0 / (t + 1.0))
            wbuf[sl] = jnp.exp(th)
        pltpu.async_copy(wbuf, s_sh.at[cols2.at[0]], ssem, add=True)

        @pl.loop(0, BLK, step=4)
        def _(i):
            wvs = [
                plsc.load_gather(wbuf, [jnp.full((LANES,), i + u, jnp.int32)])
                for u in range(4)
            ]
            for u in range(4):
                for jj in range(C // LANES):
                    xbuf[i + u, pl.ds(jj * LANES, LANES)] = (
                        xbuf[i + u, pl.ds(jj * LANES, LANES)] * wvs[u])

        pltpu.async_copy(xbuf, a_sh.at[cols2.at[0]], ssem, add=True)

    process(0, 0, drain_prev=False)
    process(1, 1, drain_prev=False)

    @pl.loop(2, NB_LO - 4, step=3)
    def _(g):
        for db in range(3):
            j = g + db
            process(j, (2 + db) % 3)

    for j in range(NB_LO - 4, NB_LO):
        process(j, j % 3)

    @pl.when(ext)
    def _():
        process(NB_LO, NB_LO % 3, prefetch=False)

    @pl.when(ext)
    def _():
        drain_scatters((NB_LO - 1) % 3)
        drain_scatters(NB_LO % 3)

    @pl.when(jnp.logical_not(ext))
    def _():
        drain_scatters((NB_LO - 2) % 3)
        drain_scatters((NB_LO - 1) % 3)

    plsc.subcore_barrier()

    for_owned_chunks(
        lambda start, size: pltpu.sync_copy(
            a_sh.at[pl.ds(start, size)], a_out.at[c, pl.ds(start, size)]))

    pltpu.sync_copy(s_sh.at[pl.ds(arow0, RPW)], zrow)
    pltpu.sync_copy(zrow, s_out.at[pl.ds(c * N + arow0, RPW)])

    @pl.when(s_ == NS - 1)
    def _():
        pltpu.sync_copy(s_sh.at[pl.ds(NS * RPW, N - NS * RPW)],
                        zrow.at[pl.ds(0, N - NS * RPW)])
        pltpu.sync_copy(zrow.at[pl.ds(0, N - NS * RPW)],
                        s_out.at[pl.ds(c * N + NS * RPW, N - NS * RPW)])


def _sc_edge_pass(x, row, col, pq):
    f = pl.kernel(
        _sc_body,
        out_type=[
            jax.ShapeDtypeStruct((NC, N, C), jnp.float32),
            jax.ShapeDtypeStruct((NC * N,), jnp.float32),
        ],
        mesh=_mesh,
        scratch_types=(
            [
                pltpu.VMEM((EPW_HI,), jnp.int32),
                pltpu.VMEM((EPW_HI,), jnp.int32),
                pltpu.VMEM((RPW,), jnp.float32),
            ]
            + [pltpu.VMEM((BLK, C), jnp.float32)] * 3
            + [pltpu.VMEM((2 * BLK,), jnp.float32)] * 3
            + [pltpu.VMEM((1, 2 * BLK), jnp.int32)] * 3
            + [pltpu.VMEM((1, BLK), jnp.int32)] * 3
            + [pltpu.VMEM((BLK,), jnp.float32)] * 3
            + [
                pltpu.VMEM_SHARED((N, C), jnp.float32),
                pltpu.VMEM_SHARED((N,), jnp.float32),
            ]
            + [pltpu.SemaphoreType.DMA] * 6
        ),
        compiler_params=_sc_compiler_params(),
    )
    return f(x, row, col, pq)


def _pq_body(w2_ref, x_ref, b_ref, o_ref):
    o_ref[...] = lax.dot_general(
        w2_ref[...], x_ref[...], (((1,), (1,)), ((), ())),
        preferred_element_type=jnp.float32) + b_ref[...]


def _pq_pass(x, w2t, bvec):
    return pl.pallas_call(
        _pq_body,
        out_shape=jax.ShapeDtypeStruct((2, N), jnp.float32),
    )(w2t, x, bvec)


def _fin_body(a_ref, s_ref, x_ref, w_ref, o_ref):
    A = a_ref[0] + a_ref[1]
    sv = jnp.sum(s_ref[...], axis=0) + 1e-16
    Y = lax.dot_general(A, w_ref[...], (((1,), (1,)), ((), ())),
                        preferred_element_type=jnp.float32)
    o_ref[...] = (1.0 - EPS) * (Y / sv[:, None]) + EPS * x_ref[...]


def _fin_pass(a_parts, s_parts, x, W_msg):
    return pl.pallas_call(
        _fin_body,
        out_shape=jax.ShapeDtypeStruct((N, C), jnp.float32),
    )(a_parts, s_parts, x, W_msg)


@jax.jit
def kernel(x, edge_index, W_att, b_att, W_msg):
    row = edge_index[0]
    col = edge_index[1]
    w2t = W_att.reshape(2, C)
    bvec = jnp.concatenate([jnp.zeros((1,), jnp.float32), b_att]).reshape(2, 1)
    pq = _pq_pass(x, w2t, bvec)
    a_parts, s_parts = _sc_edge_pass(x, row, col, pq.reshape(2 * N))
    return _fin_pass(a_parts, s_parts.reshape(NC, N), x, W_msg)
